# Initial kernel scaffold; baseline (speedup 1.0000x reference)
import jax
import jax.numpy as jnp
from jax.experimental import pallas as pl


def kernel(pos, atomic_numbers, edge_index, table, W_rbf, W_msg, W_upd, W_ii, W_ij):
    raise NotImplementedError("write your pallas kernel here")


# speedup vs baseline: 2.4238x; 2.4238x over previous
import functools
import math

import jax
import jax.numpy as jnp
import numpy as np
from jax import lax
from jax.experimental import pallas as pl
from jax.experimental.pallas import tpu as pltpu
from jax.experimental.pallas import tpu_sc as plsc

HS = 128
HBS = 32
RBF_DIM = 32
MAX_RADIUS = 15.0
NUM_LAYERS = 5
NUM_TYPES = 20
N = 10000
E = 320000

NC = 2
NS = 16
NW = NC * NS
EPW = E // NW
CH = 80
NCH = EPW // CH
RPT = N // NS

_mesh = plsc.VectorSubcoreMesh(core_axis_name="c", subcore_axis_name="s")


@functools.partial(
    pl.kernel,
    out_type=jax.ShapeDtypeStruct((E,), jnp.float32),
    mesh=_mesh,
    scratch_types=[
        pltpu.VMEM((3 * N,), jnp.float32),
        pltpu.VMEM((EPW,), jnp.int32),
        pltpu.VMEM((EPW,), jnp.int32),
        pltpu.VMEM((EPW,), jnp.float32),
    ],
    compiler_params=pltpu.CompilerParams(needs_layout_passes=False),
)
def _geom(pos_hbm, dst_hbm, src_hbm, d2_hbm, pos_v, dst_v, src_v, d2_v):
    cid = lax.axis_index("c")
    sid = lax.axis_index("s")
    base = pl.multiple_of((cid * NS + sid) * EPW, 8)
    pltpu.sync_copy(pos_hbm, pos_v)
    pltpu.sync_copy(dst_hbm.at[pl.ds(base, EPW)], dst_v)
    pltpu.sync_copy(src_hbm.at[pl.ds(base, EPW)], src_v)

    @pl.loop(0, EPW // 16)
    def _(i):
        s = pl.ds(i * 16, 16)
        vd = dst_v[s]
        vs = src_v[s]
        dx = plsc.load_gather(pos_v, [vd]) - plsc.load_gather(pos_v, [vs])
        vd = vd + N
        vs = vs + N
        dy = plsc.load_gather(pos_v, [vd]) - plsc.load_gather(pos_v, [vs])
        vd = vd + N
        vs = vs + N
        dz = plsc.load_gather(pos_v, [vd]) - plsc.load_gather(pos_v, [vs])
        d2_v[s] = dx * dx + dy * dy + dz * dz

    pltpu.sync_copy(d2_v, d2_hbm.at[pl.ds(base, EPW)])


N_PAD = 10240
RPT_P = N_PAD // NS


@functools.partial(
    pl.kernel,
    out_type=jax.ShapeDtypeStruct((NC * N_PAD, HS), jnp.float32),
    mesh=_mesh,
    scratch_types=[
        pltpu.VMEM_SHARED((N_PAD, HS), jnp.float32),
        pltpu.VMEM((CH,), jnp.int32),
        pltpu.VMEM((CH,), jnp.int32),
        pltpu.VMEM((CH, HS), jnp.float32),
        pltpu.VMEM((CH, HS), jnp.float32),
        pltpu.VMEM((128, HS), jnp.float32),
        pltpu.SemaphoreType.DMA,
    ],
    compiler_params=pltpu.CompilerParams(needs_layout_passes=False),
)
def _gms(hw_hbm, gate_hbm, dst_hbm, src_hbm, agg_hbm,
         agg_sh, dst_v, src_v, rows_v, gate_v, z_v, sem):
    cid = lax.axis_index("c")
    sid = lax.axis_index("s")

    @pl.loop(0, 128)
    def _(r):
        for j in range(HS // 16):
            z_v[r, pl.ds(j * 16, 16)] = jnp.zeros((16,), jnp.float32)

    for i in range(RPT_P // 128):
        pltpu.sync_copy(
            z_v, agg_sh.at[pl.ds(pl.multiple_of(sid * RPT_P + i * 128, 8), 128)])
    plsc.subcore_barrier()

    base_e = (cid * NS + sid) * EPW

    @pl.loop(0, NCH)
    def _(i):
        off = pl.multiple_of(base_e + i * CH, 8)
        pltpu.sync_copy(dst_hbm.at[pl.ds(off, CH)], dst_v)
        pltpu.sync_copy(src_hbm.at[pl.ds(off, CH)], src_v)
        pltpu.sync_copy(gate_hbm.at[pl.ds(off, CH)], gate_v)
        pltpu.async_copy(hw_hbm.at[src_v], rows_v, sem).wait()

        @pl.loop(0, CH)
        def _(r):
            for j in range(HS // 16):
                sl = pl.ds(j * 16, 16)
                rows_v[r, sl] = rows_v[r, sl] * gate_v[r, sl]

        pltpu.sync_copy(rows_v, agg_sh.at[dst_v], add=True)

    plsc.subcore_barrier()
    pltpu.sync_copy(agg_sh.at[pl.ds(pl.multiple_of(sid * RPT_P, 8), RPT_P)],
                    agg_hbm.at[pl.ds(pl.multiple_of(cid * N_PAD + sid * RPT_P, 8), RPT_P)])


@functools.partial(
    pl.kernel,
    out_type=jax.ShapeDtypeStruct((E, HBS), jnp.float32),
    mesh=_mesh,
    scratch_types=[
        pltpu.VMEM((CH,), jnp.int32),
        pltpu.VMEM((CH,), jnp.int32),
        pltpu.VMEM((CH, HS), jnp.float32),
        pltpu.VMEM((CH, HS), jnp.float32),
        pltpu.VMEM((CH, HBS), jnp.float32),
        pltpu.SemaphoreType.DMA,
    ],
    compiler_params=pltpu.CompilerParams(needs_layout_passes=False),
)
def _pair(hij_hbm, dst_hbm, src_hbm, out_hbm, dst_v, src_v, ra_v, rb_v, out_v, sem):
    cid = lax.axis_index("c")
    sid = lax.axis_index("s")
    base_e = (cid * NS + sid) * EPW

    @pl.loop(0, NCH)
    def _(i):
        off = pl.multiple_of(base_e + i * CH, 8)
        pltpu.sync_copy(dst_hbm.at[pl.ds(off, CH)], dst_v)
        pltpu.sync_copy(src_hbm.at[pl.ds(off, CH)], src_v)
        pltpu.async_copy(hij_hbm.at[dst_v], ra_v, sem).wait()
        pltpu.async_copy(hij_hbm.at[src_v], rb_v, sem).wait()

        @pl.loop(0, CH)
        def _(r):
            for j in range(HBS // 16):
                sl = pl.ds(j * 16, 16)
                out_v[r, sl] = ra_v[r, sl] + rb_v[r, sl]

        pltpu.sync_copy(out_v, out_hbm.at[pl.ds(off, CH)])


BN = 2000
BE = 4000


def _embed_body(an_ref, table_ref, wmsg_ref, h_ref, hw_ref):
    ids = an_ref[...]
    tt = lax.broadcasted_iota(jnp.int32, (1, NUM_TYPES), 1)
    oh = (ids == tt).astype(jnp.float32)
    h = jnp.dot(oh, table_ref[...], preferred_element_type=jnp.float32)
    h_ref[...] = h
    hw_ref[...] = jnp.dot(h, wmsg_ref[...], preferred_element_type=jnp.float32)


def _rbf_body(d2_ref, lb_ref, kv_ref, rbf_ref, cut_ref):
    d2 = d2_ref[...][:, 0]
    d = jnp.sqrt(d2 + 1e-12)
    x = jnp.exp(-0.5 * d)
    logx = jnp.maximum(-0.5 * d, math.log(1e-10))
    log1mx = jnp.log(jnp.clip(1.0 - x, 1e-10, 1.0))
    lb = lb_ref[...]
    kv = kv_ref[...]
    rbf_ref[...] = jnp.exp(lb + logx[:, None] * kv
                           + log1mx[:, None] * (float(RBF_DIM) - 1.0 - kv))
    t = jnp.clip(d / MAX_RADIUS, 0.0, 1.0)
    cut_ref[...] = (0.5 * (jnp.cos(jnp.pi * t) + 1.0))[:, None]


def _gate_body(rbf_ref, cut_ref, w_ref, gate_ref):
    pre = jnp.dot(rbf_ref[...], w_ref[...], preferred_element_type=jnp.float32)
    gate_ref[...] = pre * jax.nn.sigmoid(pre) * cut_ref[...]


def _update_body(aggA_ref, aggB_ref, h_ref, wupd_ref, wnext_ref, hn_ref, hw_ref):
    agg = aggA_ref[...] + aggB_ref[...]
    u = jnp.dot(agg, wupd_ref[...], preferred_element_type=jnp.float32)
    hn = h_ref[...] + u * jax.nn.sigmoid(u)
    hn_ref[...] = hn
    hw_ref[...] = jnp.dot(hn, wnext_ref[...], preferred_element_type=jnp.float32)


def _final_body(aggA_ref, aggB_ref, h_ref, wupd_ref, wii_ref, wij_ref,
                fii_ref, hij_ref):
    agg = aggA_ref[...] + aggB_ref[...]
    u = jnp.dot(agg, wupd_ref[...], preferred_element_type=jnp.float32)
    hn = h_ref[...] + u * jax.nn.sigmoid(u)
    fii_ref[...] = jnp.dot(hn, wii_ref[...], preferred_element_type=jnp.float32)
    hij_ref[...] = jnp.dot(hn, wij_ref[...], preferred_element_type=jnp.float32)


def _node_spec():
    return pl.BlockSpec((BN, HS), lambda i: (i, 0))


def _full(shape):
    return pl.BlockSpec(shape, lambda i: tuple(0 for _ in shape))


_embed = pl.pallas_call(
    _embed_body,
    grid=(N // BN,),
    in_specs=[pl.BlockSpec((BN, 1), lambda i: (i, 0)),
              _full((NUM_TYPES, HS)), _full((HS, HS))],
    out_specs=[_node_spec(), _node_spec()],
    out_shape=[jax.ShapeDtypeStruct((N, HS), jnp.float32),
               jax.ShapeDtypeStruct((N, HS), jnp.float32)],
)

_rbf = pl.pallas_call(
    _rbf_body,
    grid=(E // BE,),
    in_specs=[pl.BlockSpec((BE, 1), lambda i: (i, 0)),
              _full((1, RBF_DIM)), _full((1, RBF_DIM))],
    out_specs=[pl.BlockSpec((BE, RBF_DIM), lambda i: (i, 0)),
               pl.BlockSpec((BE, 1), lambda i: (i, 0))],
    out_shape=[jax.ShapeDtypeStruct((E, RBF_DIM), jnp.float32),
               jax.ShapeDtypeStruct((E, 1), jnp.float32)],
)

_gate = pl.pallas_call(
    _gate_body,
    grid=(E // BE,),
    in_specs=[pl.BlockSpec((BE, RBF_DIM), lambda i: (i, 0)),
              pl.BlockSpec((BE, 1), lambda i: (i, 0)),
              _full((RBF_DIM, HS))],
    out_specs=pl.BlockSpec((BE, HS), lambda i: (i, 0)),
    out_shape=jax.ShapeDtypeStruct((E, HS), jnp.float32),
)

_update = pl.pallas_call(
    _update_body,
    grid=(N // BN,),
    in_specs=[_node_spec(), _node_spec(), _node_spec(),
              _full((HS, HS)), _full((HS, HS))],
    out_specs=[_node_spec(), _node_spec()],
    out_shape=[jax.ShapeDtypeStruct((N, HS), jnp.float32),
               jax.ShapeDtypeStruct((N, HS), jnp.float32)],
)

_final = pl.pallas_call(
    _final_body,
    grid=(N // BN,),
    in_specs=[_node_spec(), _node_spec(), _node_spec(),
              _full((HS, HS)), _full((HS, HBS)), _full((HS, HS))],
    out_specs=[pl.BlockSpec((BN, HBS), lambda i: (i, 0)),
               pl.BlockSpec((BN, HS), lambda i: (i, 0))],
    out_shape=[jax.ShapeDtypeStruct((N, HBS), jnp.float32),
               jax.ShapeDtypeStruct((N, HS), jnp.float32)],
)

_LOGBINOM = np.array(
    [[math.lgamma(RBF_DIM) - math.lgamma(k + 1.0) - math.lgamma(RBF_DIM - k)
      for k in range(RBF_DIM)]], dtype=np.float32)
_KVEC = np.arange(RBF_DIM, dtype=np.float32)[None, :]


def kernel(pos, atomic_numbers, edge_index, table, W_rbf, W_msg, W_upd, W_ii, W_ij):
    dst = edge_index[0].astype(jnp.int32)
    src = edge_index[1].astype(jnp.int32)
    an2 = atomic_numbers.astype(jnp.int32).reshape(N, 1)

    d2 = _geom(pos.T.reshape(-1), dst, src)
    rbf, cut = _rbf(d2.reshape(E, 1), jnp.asarray(_LOGBINOM), jnp.asarray(_KVEC))

    h, hw = _embed(an2, table, W_msg[0])
    for l in range(NUM_LAYERS):
        gate = _gate(rbf, cut, W_rbf[l])
        aggp = _gms(hw, gate, dst, src)
        aggA, aggB = aggp[:N], aggp[N_PAD:N_PAD + N]
        if l < NUM_LAYERS - 1:
            h, hw = _update(aggA, aggB, h, W_upd[l], W_msg[l + 1])
        else:
            wij_pad = jnp.pad(W_ij, ((0, 0), (0, HS - HBS)))
            fii, hij = _final(aggA, aggB, h, W_upd[l], W_ii, wij_pad)

    fij = _pair(hij, dst, src)
    return jnp.concatenate([fii, fij], axis=0)
